# baseline (speedup 1.0000x reference)
import jax
import jax.numpy as jnp
from jax.experimental import pallas as pl

N_TOKENS = 32768
D_EMBED = 768
N_EXPERTS = 64
STREAMS = 8
ROWS = 512
STEP = STREAMS * ROWS


def _router_block(*refs):
    x_refs = refs[:STREAMS]
    wt_ref, b_ref = refs[STREAMS], refs[STREAMS + 1]
    scores_ref, idx_ref = refs[STREAMS + 2], refs[STREAMS + 3]
    wt = wt_ref[...]
    bias = b_ref[...]
    for k in range(STREAMS):
        logits = jnp.dot(x_refs[k][...], wt, preferred_element_type=jnp.float32)
        logits = logits + bias
        lt = logits.T

        eid = jax.lax.broadcasted_iota(jnp.int32, lt.shape, 0).astype(jnp.float32)
        m1 = jnp.max(lt, axis=0, keepdims=True)
        i1f = jnp.min(jnp.where(lt == m1, eid, 64.0), axis=0, keepdims=True)
        lt2 = jnp.where(eid == i1f, -jnp.inf, lt)
        m2 = jnp.max(lt2, axis=0, keepdims=True)
        i2f = jnp.min(jnp.where(lt2 == m2, eid, 64.0), axis=0, keepdims=True)

        denom = jnp.sum(jnp.exp(lt - m1), axis=0, keepdims=True)
        s1 = 1.0 / denom
        s2 = jnp.exp(m2 - m1) / denom

        cols = pl.ds(k * ROWS, ROWS)
        scores_ref[:, cols] = jnp.concatenate([s1, s2], axis=0)
        idx_ref[:, cols] = jnp.concatenate([i1f, i2f], axis=0).astype(jnp.int32)


@jax.jit
def kernel(x, W, b):
    wt = W.T
    b2 = b.reshape(1, N_EXPERTS)
    grid = (N_TOKENS // STEP,)
    in_specs = [
        pl.BlockSpec((ROWS, D_EMBED), lambda i, k=k: (STREAMS * i + k, 0))
        for k in range(STREAMS)
    ] + [
        pl.BlockSpec((D_EMBED, N_EXPERTS), lambda i: (0, 0)),
        pl.BlockSpec((1, N_EXPERTS), lambda i: (0, 0)),
    ]
    scores_t, idx_t = pl.pallas_call(
        _router_block,
        grid=grid,
        in_specs=in_specs,
        out_specs=[
            pl.BlockSpec((2, STEP), lambda i: (0, i)),
            pl.BlockSpec((2, STEP), lambda i: (0, i)),
        ],
        out_shape=[
            jax.ShapeDtypeStruct((2, N_TOKENS), jnp.float32),
            jax.ShapeDtypeStruct((2, N_TOKENS), jnp.int32),
        ],
    )(*([x] * STREAMS + [wt, b2]))
    return scores_t.T, idx_t.T
